# baseline (speedup 1.0000x reference)
import jax
import jax.numpy as jnp
from jax.experimental import pallas as pl
from jax.experimental.pallas import tpu as pltpu


def _fused_kernel(x_ref, wp_ref, bp_ref, w1_ref, b1_ref, g1_ref, be1_ref,
                  w2_ref, b2_ref, g2_ref, be2_ref, w3_ref, b3_ref,
                  out_ref, x16_ref, wp16_ref, h1_ref, idx_ref,
                  acc0_ref, acc1_ref):
    i = pl.program_id(0)
    nblk = pl.num_programs(0) // 3
    tb = x_ref.shape[1]
    dch = wp_ref.shape[1]
    d = x_ref.shape[2]
    dh = d // 2

    @pl.when((i < 2 * nblk) & (i % 2 == 0))
    def _stage_wp():
        c = i // 2
        wp16_ref[:, pl.ds(c * dch, dch), :] = wp_ref[...].astype(jnp.bfloat16)

    @pl.when(i < nblk)
    def _classify_step():
        x16 = x_ref[0].astype(jnp.bfloat16)
        x16_ref[pl.ds(i * tb, tb), :] = x16
        h1 = jnp.dot(x16, w1_ref[...].astype(jnp.bfloat16),
                     preferred_element_type=jnp.float32)
        h1_ref[pl.ds(i * tb, tb), :] = h1

    @pl.when(i == nblk - 1)
    def _finish_classifier():
        h = h1_ref[...] + b1_ref[...]
        mu = jnp.mean(h, axis=0, keepdims=True)
        var = jnp.mean((h - mu) ** 2, axis=0, keepdims=True)
        h = (h - mu) / jnp.sqrt(var + 1e-5) * g1_ref[...] + be1_ref[...]
        h = jnp.maximum(h, 0.0)
        h = jnp.dot(h.astype(jnp.bfloat16), w2_ref[...].astype(jnp.bfloat16),
                    preferred_element_type=jnp.float32)
        h = h + b2_ref[...]
        mu = jnp.mean(h, axis=0, keepdims=True)
        var = jnp.mean((h - mu) ** 2, axis=0, keepdims=True)
        h = (h - mu) / jnp.sqrt(var + 1e-5) * g2_ref[...] + be2_ref[...]
        h = jnp.maximum(h, 0.0)
        h16 = h.astype(jnp.bfloat16).astype(jnp.float32)
        w3 = w3_ref[...].astype(jnp.bfloat16).astype(jnp.float32)
        v = jnp.sum(h16 * w3, axis=1, keepdims=True) + b3_ref[...]
        z = jax.nn.sigmoid(v)
        idx_ref[...] = jnp.clip(jnp.round(z), 0.0, 1.0).astype(jnp.int32)

    @pl.when((i >= nblk) & (i < 2 * nblk))
    def _head_first_half():
        j = i - nblk
        xb = x16_ref[pl.ds(j * tb, tb), 0:dh]
        acc0_ref[pl.ds(j * tb, tb), :] = jnp.dot(
            xb, wp16_ref[0, 0:dh, :], preferred_element_type=jnp.float32)
        acc1_ref[pl.ds(j * tb, tb), :] = jnp.dot(
            xb, wp16_ref[1, 0:dh, :], preferred_element_type=jnp.float32)

    @pl.when(i >= 2 * nblk)
    def _head_second_half_select():
        j = i - 2 * nblk
        xb = x16_ref[pl.ds(j * tb, tb), dh:d]
        o0 = acc0_ref[pl.ds(j * tb, tb), :] + jnp.dot(
            xb, wp16_ref[0, dh:d, :], preferred_element_type=jnp.float32)
        o1 = acc1_ref[pl.ds(j * tb, tb), :] + jnp.dot(
            xb, wp16_ref[1, dh:d, :], preferred_element_type=jnp.float32)
        m = idx_ref[pl.ds(j * tb, tb), :] > 0
        out_ref[0] = jnp.where(m, o1 + bp_ref[1:2, :], o0 + bp_ref[0:1, :])


def kernel(x, W1, b1, g1, be1, W2, b2, g2, be2, W3, b3, Wp, bp):
    Bx, Nx, D = x.shape
    T = Bx * Nx
    C, _, P = Wp.shape
    H1 = W1.shape[1]
    TB = 256
    nblk = T // TB
    DCH = D // nblk
    nb = Nx // TB

    def _xmap(i):
        j = jnp.minimum(i, nblk - 1)
        return (j // nb, j % nb, 0)

    def _wpmap(i):
        return (0, jnp.minimum(i, 2 * nblk - 1) // 2, 0)

    def _omap(i):
        j = jnp.maximum(i - 2 * nblk, 0)
        return (j // nb, j % nb, 0)

    out = pl.pallas_call(
        _fused_kernel,
        grid=(3 * nblk,),
        in_specs=[
            pl.BlockSpec((1, TB, D), _xmap),
            pl.BlockSpec((C, DCH, P), _wpmap),
            pl.BlockSpec((C, P), lambda i: (0, 0)),
            pl.BlockSpec((D, H1), lambda i: (0, 0)),
        ] + [pl.BlockSpec(None, lambda i: (0, 0))] * 9,
        out_specs=pl.BlockSpec((1, TB, P), _omap),
        out_shape=jax.ShapeDtypeStruct((Bx, Nx, P), jnp.float32),
        scratch_shapes=[
            pltpu.VMEM((T, D), jnp.bfloat16),
            pltpu.VMEM((C, D, P), jnp.bfloat16),
            pltpu.VMEM((T, H1), jnp.float32),
            pltpu.VMEM((T, 1), jnp.int32),
            pltpu.VMEM((T, P), jnp.float32),
            pltpu.VMEM((T, P), jnp.float32),
        ],
    )(x, Wp, bp, W1, b1.reshape(1, -1), g1.reshape(1, -1), be1.reshape(1, -1),
      W2, b2.reshape(1, -1), g2.reshape(1, -1), be2.reshape(1, -1),
      W3.reshape(1, -1), b3.reshape(1, -1))

    return out


# speedup vs baseline: 1.0669x; 1.0669x over previous
import jax
import jax.numpy as jnp
from jax.experimental import pallas as pl
from jax.experimental.pallas import tpu as pltpu


def _fused_kernel(x_ref, wp_ref, w1_ref, b1_ref, g1_ref, be1_ref,
                  w2_ref, b2_ref, g2_ref, be2_ref, w3_ref, b3_ref, bp_ref,
                  out_ref, x16_ref, wp16_ref, h1_ref, idx_ref):
    i = pl.program_id(0)
    nblk = pl.num_programs(0) // 2
    tb = x_ref.shape[1]
    dchunk = wp_ref.shape[1]

    @pl.when(i < nblk)
    def _classify_step():
        x16 = x_ref[0].astype(jnp.bfloat16)
        x16_ref[pl.ds(i * tb, tb), :] = x16
        wp16_ref[:, pl.ds(i * dchunk, dchunk), :] = wp_ref[...].astype(jnp.bfloat16)
        h1 = jnp.dot(x16, w1_ref[...].astype(jnp.bfloat16),
                     preferred_element_type=jnp.float32)
        h1_ref[pl.ds(i * tb, tb), :] = h1

    @pl.when(i == nblk - 1)
    def _finish_classifier():
        h = h1_ref[...] + b1_ref[...]
        mu = jnp.mean(h, axis=0, keepdims=True)
        var = jnp.mean((h - mu) ** 2, axis=0, keepdims=True)
        h = (h - mu) / jnp.sqrt(var + 1e-5) * g1_ref[...] + be1_ref[...]
        h = jnp.maximum(h, 0.0)
        h = jnp.dot(h.astype(jnp.bfloat16), w2_ref[...].astype(jnp.bfloat16),
                    preferred_element_type=jnp.float32)
        h = h + b2_ref[...]
        mu = jnp.mean(h, axis=0, keepdims=True)
        var = jnp.mean((h - mu) ** 2, axis=0, keepdims=True)
        h = (h - mu) / jnp.sqrt(var + 1e-5) * g2_ref[...] + be2_ref[...]
        h = jnp.maximum(h, 0.0)
        h16 = h.astype(jnp.bfloat16).astype(jnp.float32)
        w3 = w3_ref[...].astype(jnp.bfloat16).astype(jnp.float32)
        v = jnp.sum(h16 * w3, axis=1, keepdims=True) + b3_ref[...]
        z = jax.nn.sigmoid(v)
        idx_ref[...] = jnp.clip(jnp.round(z), 0.0, 1.0).astype(jnp.int32)

    @pl.when(i >= nblk)
    def _head_step():
        j = i - nblk
        xb = x16_ref[pl.ds(j * tb, tb), :]
        o0 = jnp.dot(xb, wp16_ref[0], preferred_element_type=jnp.float32)
        o1 = jnp.dot(xb, wp16_ref[1], preferred_element_type=jnp.float32)
        m = (idx_ref[pl.ds(j * tb, tb), :] > 0)
        out_ref[0] = jnp.where(m, o1 + bp_ref[1:2, :], o0 + bp_ref[0:1, :])


def kernel(x, W1, b1, g1, be1, W2, b2, g2, be2, W3, b3, Wp, bp):
    Bx, Nx, D = x.shape
    T = Bx * Nx
    C, _, P = Wp.shape
    H1 = W1.shape[1]
    TB = 256
    nblk = T // TB
    DCH = D // nblk
    nb = Nx // TB

    def _xmap(i):
        j = jnp.minimum(i, nblk - 1)
        return (j // nb, j % nb, 0)

    def _omap(i):
        j = jnp.maximum(i - nblk, 0)
        return (j // nb, j % nb, 0)

    out = pl.pallas_call(
        _fused_kernel,
        grid=(2 * nblk,),
        in_specs=[
            pl.BlockSpec((1, TB, D), _xmap),
            pl.BlockSpec((C, DCH, P), lambda i: (0, jnp.minimum(i, nblk - 1), 0)),
            pl.BlockSpec((D, H1), lambda i: (0, 0)),
        ] + [pl.BlockSpec(None, lambda i: (0, 0))] * 10,
        out_specs=pl.BlockSpec((1, TB, P), _omap),
        out_shape=jax.ShapeDtypeStruct((Bx, Nx, P), jnp.float32),
        scratch_shapes=[
            pltpu.VMEM((T, D), jnp.bfloat16),
            pltpu.VMEM((C, D, P), jnp.bfloat16),
            pltpu.VMEM((T, H1), jnp.float32),
            pltpu.VMEM((T, 1), jnp.int32),
        ],
    )(x, Wp, W1, b1.reshape(1, -1), g1.reshape(1, -1), be1.reshape(1, -1),
      W2, b2.reshape(1, -1), g2.reshape(1, -1), be2.reshape(1, -1),
      W3.reshape(1, -1), b3.reshape(1, -1), bp)

    return out
